# baseline (speedup 1.0000x reference)
import functools

import jax
import jax.numpy as jnp
from jax import lax
from jax.experimental import pallas as pl
from jax.experimental.pallas import tpu as pltpu
from jax.experimental.pallas import tpu_sc as plsc

_NC = 2
_NS = 16
_NW = _NC * _NS
_L = 16
_CH = 1024


def _sc_body(bpw, d, k, z_hbm, a_hbm, out_hbm, zbuf, abuf, obuf, zsem, asem, osem):
    wid = lax.axis_index("s") * _NC + lax.axis_index("c")
    base = wid * bpw
    kf = jnp.float32(k)
    kmax = jnp.int32(k - 1)
    nch = bpw // _CH

    def zcopy(ci, b):
        return pltpu.make_async_copy(
            z_hbm.at[pl.ds(base + ci * _CH, _CH), :],
            zbuf.at[pl.ds(b * _CH, _CH), :],
            zsem,
        )

    def acopy(ci, b):
        return pltpu.make_async_copy(
            a_hbm.at[pl.ds(base + ci * _CH, _CH), :],
            abuf.at[pl.ds(b * _CH, _CH), :],
            asem,
        )

    def odrain(b):
        return pltpu.make_async_copy(
            obuf.at[pl.ds(b * _CH, _CH)], out_hbm.at[pl.ds(base, _CH)], osem
        )

    zcopy(0, 0).start()
    acopy(0, 0).start()

    def chunk_work(ci, b):
        cbase = base + ci * _CH

        @pl.when(ci >= 2)
        def _():
            odrain(b).wait()

        zcopy(ci, b).wait()
        acopy(ci, b).wait()

        @pl.when(ci + 1 < nch)
        def _():
            zcopy(ci + 1, 1 - b).start()
            acopy(ci + 1, 1 - b).start()

        zeros = jnp.zeros((_L,), jnp.int32)

        def step(v, _):
            r0 = b * _CH + v * _L
            rows = lax.iota(jnp.int32, _L) + r0
            zc = plsc.load_gather(zbuf, [rows, zeros])
            idx = jnp.clip((zc * kf).astype(jnp.int32), 0, kmax)
            picked = plsc.load_gather(abuf, [rows, idx])
            obuf[pl.ds(r0, _L)] = picked * 0.999
            return 0

        lax.fori_loop(0, _CH // _L, step, 0)
        pltpu.make_async_copy(
            obuf.at[pl.ds(b * _CH, _CH)], out_hbm.at[pl.ds(cbase, _CH)], osem
        ).start()

    def loop_body(i, _):
        chunk_work(2 * i, 0)
        chunk_work(2 * i + 1, 1)
        return 0

    lax.fori_loop(0, nch // 2, loop_body, 0)
    odrain(0).wait()
    odrain(1).wait()


def kernel(z, a):
    b, d = z.shape
    _, k = a.shape
    bpw = b // _NW
    assert b % (_NW * _CH * 2) == 0
    mesh = plsc.VectorSubcoreMesh(
        core_axis_name="c", subcore_axis_name="s", num_cores=_NC, num_subcores=_NS
    )
    fn = pl.kernel(
        functools.partial(_sc_body, bpw, d, k),
        out_type=jax.ShapeDtypeStruct((b,), jnp.float32),
        mesh=mesh,
        compiler_params=pltpu.CompilerParams(
            needs_layout_passes=False, use_tc_tiling_on_sc=False
        ),
        scratch_types=[
            pltpu.VMEM((2 * _CH, d), jnp.float32),
            pltpu.VMEM((2 * _CH, k), jnp.float32),
            pltpu.VMEM((2 * _CH,), jnp.float32),
            pltpu.SemaphoreType.DMA,
            pltpu.SemaphoreType.DMA,
            pltpu.SemaphoreType.DMA,
        ],
    )
    return fn(z, a)
